# Initial kernel scaffold; baseline (speedup 1.0000x reference)
import jax
import jax.numpy as jnp
from jax.experimental import pallas as pl


def kernel(boxes, box_scores):
    raise NotImplementedError("write your pallas kernel here")


# speedup vs baseline: 121.5140x; 121.5140x over previous
import functools

import jax
import jax.numpy as jnp
from jax.experimental import pallas as pl
from jax.experimental.pallas import tpu as pltpu

_N = 5000
_C = 80
_P = 5120
_STRIP = 32
_NMS_THRESH = 0.5
_SCORE_THRESH = 0.05
_MAX_H = 15
_MAX_O = 15
_NEG = -1e9


def _body(boxes_ref, boxes_t_ref, cls_ref, cls_t_ref, out_ref,
          t_ref, colbuf_ref, keepcol_ref):
    f32 = jnp.float32

    m = jnp.max(boxes_ref[...])
    off_scale = m + 1.0

    i32 = jnp.int32
    big = i32(2**30)
    cls = cls_ref[...]
    scr_c = jnp.max(cls, axis=1, keepdims=True)
    li = jax.lax.broadcasted_iota(i32, (_P, _C), 1)
    lbl_c = jnp.min(jnp.where(cls == scr_c, li, big), axis=1, keepdims=True)
    off_c = off_scale * lbl_c.astype(f32)
    x1c = boxes_ref[:, 0:1] + off_c
    y1c = boxes_ref[:, 1:2] + off_c
    x2c = boxes_ref[:, 2:3] + off_c
    y2c = boxes_ref[:, 3:4] + off_c
    area_c = (x2c - x1c) * (y2c - y1c)
    colbuf_ref[:, 0:1] = x1c
    colbuf_ref[:, 1:2] = y1c
    colbuf_ref[:, 2:3] = x2c
    colbuf_ref[:, 3:4] = y2c
    colbuf_ref[:, 4:5] = area_c
    colbuf_ref[:, 5:6] = scr_c
    colbuf_ref[:, 6:7] = jnp.zeros((_P, 1), f32)
    colbuf_ref[:, 7:8] = jnp.zeros((_P, 1), f32)

    cls_t = cls_t_ref[...]
    scr_r = jnp.max(cls_t, axis=0, keepdims=True)
    si = jax.lax.broadcasted_iota(i32, (_C, _P), 0)
    lbl_r = jnp.min(jnp.where(cls_t == scr_r, si, big), axis=0, keepdims=True)
    lbl_rf = lbl_r.astype(f32)
    off_r = off_scale * lbl_rf
    x1r = boxes_t_ref[0:1, :] + off_r
    y1r = boxes_t_ref[1:2, :] + off_r
    x2r = boxes_t_ref[2:3, :] + off_r
    y2r = boxes_t_ref[3:4, :] + off_r
    area_r = (x2r - x1r) * (y2r - y1r)
    idx_r = jax.lax.broadcasted_iota(i32, (1, _P), 1)

    def strip(ib, _):
        base = ib * _STRIP
        cb = colbuf_ref[pl.ds(base, _STRIP), :]
        x1o = cb[:, 0:1]
        y1o = cb[:, 1:2]
        x2o = cb[:, 2:3]
        y2o = cb[:, 3:4]
        ao = cb[:, 4:5]
        so = cb[:, 5:6]
        io = base + jax.lax.broadcasted_iota(i32, (_STRIP, 1), 0)
        ltx = jnp.maximum(x1o, x1r)
        lty = jnp.maximum(y1o, y1r)
        rbx = jnp.minimum(x2o, x2r)
        rby = jnp.minimum(y2o, y2r)
        w = jnp.maximum(rbx - ltx, 0.0)
        h = jnp.maximum(rby - lty, 0.0)
        inter = w * h
        iou = inter / (ao + area_r - inter + 1e-9)
        rank = (so > scr_r) | ((so == scr_r) & (io < idx_r))
        t = ((iou > _NMS_THRESH) & rank).astype(jnp.int8)
        t_ref[pl.ds(base, _STRIP), :] = t
        return 0

    jax.lax.fori_loop(0, _P // _STRIP, strip, 0)

    def cond(carry):
        return carry[1]

    def body(carry):
        keep, _ = carry
        keepcol_ref[...] = jnp.transpose(keep)

        def chunk(c, sup):
            tc = t_ref[pl.ds(c * 128, 128), :].astype(f32)
            kc = keepcol_ref[pl.ds(c * 128, 128), :]
            return jnp.maximum(sup, jnp.max(tc * kc, axis=0, keepdims=True))

        sup = jax.lax.fori_loop(0, _P // 128, chunk, jnp.zeros((1, _P), f32))
        keep_new = (sup == 0.0).astype(f32)
        changed = jnp.any(keep_new != keep)
        return keep_new, changed

    keep, _ = jax.lax.while_loop(
        cond, body, (jnp.ones((1, _P), f32), jnp.bool_(True)))

    kept = (keep > 0.0) & (scr_r >= _SCORE_THRESH)
    is_h = lbl_r == 0
    neg_inf = f32(-jnp.inf)
    prio_h = jnp.where(kept & is_h, scr_r, neg_inf)
    prio_o = jnp.where(kept & (~is_h), scr_r, neg_inf)

    bx1 = boxes_t_ref[0:1, :]
    by1 = boxes_t_ref[1:2, :]
    bx2 = boxes_t_ref[2:3, :]
    by2 = boxes_t_ref[3:4, :]
    lane = jax.lax.broadcasted_iota(jnp.int32, (1, 128), 1)

    out_ref[...] = jnp.zeros((32, 128), f32)

    def emit(slot, prio):
        mval = jnp.max(prio)
        sel = jnp.min(jnp.where(prio == mval, idx_r, big))
        onehot = idx_r == sel
        vf = (mval > neg_inf).astype(f32)
        gx1 = jnp.max(jnp.where(onehot, bx1, neg_inf)) * vf
        gy1 = jnp.max(jnp.where(onehot, by1, neg_inf)) * vf
        gx2 = jnp.max(jnp.where(onehot, bx2, neg_inf)) * vf
        gy2 = jnp.max(jnp.where(onehot, by2, neg_inf)) * vf
        gsc = jnp.max(jnp.where(onehot, scr_r, neg_inf)) * vf
        glb = jnp.max(jnp.where(onehot, lbl_rf, neg_inf))
        glb = jnp.where(mval > neg_inf, glb, f32(-1.0))
        gid = jnp.where(mval > neg_inf, sel.astype(f32), f32(-1.0))
        row = jnp.where(lane == 0, gx1,
              jnp.where(lane == 1, gy1,
              jnp.where(lane == 2, gx2,
              jnp.where(lane == 3, gy2,
              jnp.where(lane == 4, gsc,
              jnp.where(lane == 5, glb,
              jnp.where(lane == 6, gid, f32(0.0))))))))
        out_ref[pl.ds(slot, 1), :] = row
        return jnp.where(onehot, neg_inf, prio)

    for k in range(_MAX_H):
        prio_h = emit(k, prio_h)
    for k in range(_MAX_O):
        prio_o = emit(_MAX_H + k, prio_o)


@functools.partial(jax.jit)
def kernel(boxes, box_scores):
    f32 = jnp.float32
    boxes_p = jnp.full((_P, 4), _NEG, f32).at[:_N, :].set(boxes)
    cls_p = jnp.full((_P, _C), -jnp.inf, f32).at[:_N, :].set(
        box_scores[:, :_C])
    boxes_t = boxes_p.T
    cls_t = cls_p.T

    out = pl.pallas_call(
        _body,
        out_shape=jax.ShapeDtypeStruct((32, 128), f32),
        scratch_shapes=[
            pltpu.VMEM((_P, _P), jnp.int8),
            pltpu.VMEM((_P, 8), f32),
            pltpu.VMEM((_P, 1), f32),
        ],
    )(boxes_p, boxes_t, cls_p, cls_t)

    coords = out[:30, 0:4]
    out_scores = out[:30, 4]
    out_labels = out[:30, 5].astype(jnp.int32)
    out_idx = out[:30, 6].astype(jnp.int32)
    return coords, out_scores, out_labels, out_idx


# speedup vs baseline: 129.0950x; 1.0624x over previous
import functools

import jax
import jax.numpy as jnp
from jax.experimental import pallas as pl
from jax.experimental.pallas import tpu as pltpu

_N = 5000
_C = 80
_P = 5120
_STRIP = 32
_NMS_THRESH = 0.5
_SCORE_THRESH = 0.05
_MAX_H = 15
_MAX_O = 15
_NEG = -1e9


def _body(boxes_ref, boxes_t_ref, cls_ref, cls_t_ref, out_ref,
          t_ref, colbuf_ref, keepcol_ref):
    f32 = jnp.float32

    m = jnp.max(boxes_ref[...])
    off_scale = m + 1.0

    i32 = jnp.int32
    big = i32(2**30)
    cls = cls_ref[...]
    scr_c = jnp.max(cls, axis=1, keepdims=True)
    li = jax.lax.broadcasted_iota(i32, (_P, _C), 1)
    lbl_c = jnp.min(jnp.where(cls == scr_c, li, big), axis=1, keepdims=True)
    off_c = off_scale * lbl_c.astype(f32)
    x1c = boxes_ref[:, 0:1] + off_c
    y1c = boxes_ref[:, 1:2] + off_c
    x2c = boxes_ref[:, 2:3] + off_c
    y2c = boxes_ref[:, 3:4] + off_c
    area_c = (x2c - x1c) * (y2c - y1c)
    colbuf_ref[:, 0:1] = x1c
    colbuf_ref[:, 1:2] = y1c
    colbuf_ref[:, 2:3] = x2c
    colbuf_ref[:, 3:4] = y2c
    colbuf_ref[:, 4:5] = area_c
    colbuf_ref[:, 5:6] = scr_c
    colbuf_ref[:, 6:7] = jnp.zeros((_P, 1), f32)
    colbuf_ref[:, 7:8] = jnp.zeros((_P, 1), f32)

    cls_t = cls_t_ref[...]
    scr_r = jnp.max(cls_t, axis=0, keepdims=True)
    si = jax.lax.broadcasted_iota(i32, (_C, _P), 0)
    lbl_r = jnp.min(jnp.where(cls_t == scr_r, si, big), axis=0, keepdims=True)
    lbl_rf = lbl_r.astype(f32)
    off_r = off_scale * lbl_rf
    x1r = boxes_t_ref[0:1, :] + off_r
    y1r = boxes_t_ref[1:2, :] + off_r
    x2r = boxes_t_ref[2:3, :] + off_r
    y2r = boxes_t_ref[3:4, :] + off_r
    area_r = (x2r - x1r) * (y2r - y1r)
    idx_r = jax.lax.broadcasted_iota(i32, (1, _P), 1)

    def strip(ib, _):
        base = ib * _STRIP
        cb = colbuf_ref[pl.ds(base, _STRIP), :]
        x1o = cb[:, 0:1]
        y1o = cb[:, 1:2]
        x2o = cb[:, 2:3]
        y2o = cb[:, 3:4]
        ao = cb[:, 4:5]
        so = cb[:, 5:6]
        io = base + jax.lax.broadcasted_iota(i32, (_STRIP, 1), 0)
        ltx = jnp.maximum(x1o, x1r)
        lty = jnp.maximum(y1o, y1r)
        rbx = jnp.minimum(x2o, x2r)
        rby = jnp.minimum(y2o, y2r)
        w = jnp.maximum(rbx - ltx, 0.0)
        h = jnp.maximum(rby - lty, 0.0)
        inter = w * h
        denom = ao + area_r - inter + 1e-9
        rank = (so > scr_r) | ((so == scr_r) & (io < idx_r))
        t = ((inter > _NMS_THRESH * denom) & rank).astype(jnp.int8)
        t_ref[pl.ds(base, _STRIP), :] = t
        return 0

    jax.lax.fori_loop(0, _P // _STRIP, strip, 0)

    def cond(carry):
        return carry[1]

    def body(carry):
        keep, _ = carry
        keepcol_ref[...] = jnp.transpose(keep).astype(jnp.int8)

        def chunk(c, sup):
            tc = t_ref[pl.ds(c * 128, 128), :]
            kc = keepcol_ref[pl.ds(c * 128, 128), :]
            m = (tc & kc).astype(f32)
            return jnp.maximum(sup, jnp.max(m, axis=0, keepdims=True))

        sup = jax.lax.fori_loop(0, _P // 128, chunk, jnp.zeros((1, _P), f32))
        keep_new = (sup == 0.0).astype(f32)
        changed = jnp.any(keep_new != keep)
        return keep_new, changed

    keep, _ = jax.lax.while_loop(
        cond, body, (jnp.ones((1, _P), f32), jnp.bool_(True)))

    kept = (keep > 0.0) & (scr_r >= _SCORE_THRESH)
    is_h = lbl_r == 0
    neg_inf = f32(-jnp.inf)
    prio_h = jnp.where(kept & is_h, scr_r, neg_inf)
    prio_o = jnp.where(kept & (~is_h), scr_r, neg_inf)

    bx1 = boxes_t_ref[0:1, :]
    by1 = boxes_t_ref[1:2, :]
    bx2 = boxes_t_ref[2:3, :]
    by2 = boxes_t_ref[3:4, :]
    lane = jax.lax.broadcasted_iota(jnp.int32, (1, 128), 1)

    out_ref[...] = jnp.zeros((32, 128), f32)

    def emit(slot, prio):
        mval = jnp.max(prio)
        sel = jnp.min(jnp.where(prio == mval, idx_r, big))
        onehot = idx_r == sel
        vf = (mval > neg_inf).astype(f32)
        gx1 = jnp.max(jnp.where(onehot, bx1, neg_inf)) * vf
        gy1 = jnp.max(jnp.where(onehot, by1, neg_inf)) * vf
        gx2 = jnp.max(jnp.where(onehot, bx2, neg_inf)) * vf
        gy2 = jnp.max(jnp.where(onehot, by2, neg_inf)) * vf
        gsc = jnp.max(jnp.where(onehot, scr_r, neg_inf)) * vf
        glb = jnp.max(jnp.where(onehot, lbl_rf, neg_inf))
        glb = jnp.where(mval > neg_inf, glb, f32(-1.0))
        gid = jnp.where(mval > neg_inf, sel.astype(f32), f32(-1.0))
        row = jnp.where(lane == 0, gx1,
              jnp.where(lane == 1, gy1,
              jnp.where(lane == 2, gx2,
              jnp.where(lane == 3, gy2,
              jnp.where(lane == 4, gsc,
              jnp.where(lane == 5, glb,
              jnp.where(lane == 6, gid, f32(0.0))))))))
        out_ref[pl.ds(slot, 1), :] = row
        return jnp.where(onehot, neg_inf, prio)

    for k in range(_MAX_H):
        prio_h = emit(k, prio_h)
    for k in range(_MAX_O):
        prio_o = emit(_MAX_H + k, prio_o)


@functools.partial(jax.jit)
def kernel(boxes, box_scores):
    f32 = jnp.float32
    boxes_p = jnp.full((_P, 4), _NEG, f32).at[:_N, :].set(boxes)
    cls_p = jnp.full((_P, _C), -jnp.inf, f32).at[:_N, :].set(
        box_scores[:, :_C])
    boxes_t = boxes_p.T
    cls_t = cls_p.T

    out = pl.pallas_call(
        _body,
        out_shape=jax.ShapeDtypeStruct((32, 128), f32),
        scratch_shapes=[
            pltpu.VMEM((_P, _P), jnp.int8),
            pltpu.VMEM((_P, 8), f32),
            pltpu.VMEM((_P, 1), jnp.int8),
        ],
    )(boxes_p, boxes_t, cls_p, cls_t)

    coords = out[:30, 0:4]
    out_scores = out[:30, 4]
    out_labels = out[:30, 5].astype(jnp.int32)
    out_idx = out[:30, 6].astype(jnp.int32)
    return coords, out_scores, out_labels, out_idx
